# baseline (speedup 1.0000x reference)
import functools

import jax
import jax.numpy as jnp
from jax import lax
from jax.experimental import pallas as pl
from jax.experimental.pallas import tpu as pltpu
from jax.experimental.pallas import tpu_sc as plsc

N = 10000
E = 320000
D = 128
G = 64
OUT = 64
STEPS = 4
LSTM_LAYERS = 4

NC = 2
NS = 16
NW = NC * NS
CHUNK = 128
EPW_CHUNKS = 80
EPAD = NW * EPW_CHUNKS * CHUNK
NPAD = 10112


def _sc_edge_agg(x, src_r, dst_r, zeros_init):
    mesh = plsc.VectorSubcoreMesh(core_axis_name="c", subcore_axis_name="s")
    stripe = NPAD // NS
    NBUF = 2
    SUBC_CHUNKS = EPAD // CHUNK // NS
    STAGE = SUBC_CHUNKS // 4

    @functools.partial(
        pl.kernel,
        out_type=jax.ShapeDtypeStruct((NPAD, D), jnp.float32),
        mesh=mesh,
        scratch_types=[
            pltpu.VMEM((STAGE, CHUNK), jnp.int32),
            pltpu.VMEM((STAGE, CHUNK), jnp.int32),
            [pltpu.VMEM((CHUNK, D), jnp.float32) for _ in range(NBUF)],
            pltpu.VMEM_SHARED((NPAD, D), jnp.float32),
            pltpu.SemaphoreType.DMA((NBUF,)),
        ],
    )
    def agg_kernel(x_hbm, src_hbm, dst_hbm, z_hbm, out_hbm,
                   src_v, dst_v, rows, acc_sh, gsem):
        c = lax.axis_index("c")
        s = lax.axis_index("s")

        @pl.when(c == 0)
        def _():
            pltpu.sync_copy(z_hbm.at[pl.ds(s * stripe, stripe)],
                            acc_sh.at[pl.ds(s * stripe, stripe)])
            plsc.subcore_barrier()

            for stage in range(SUBC_CHUNKS // STAGE):
                pltpu.sync_copy(src_hbm.at[s, pl.ds(stage * STAGE, STAGE)],
                                src_v)
                pltpu.sync_copy(dst_hbm.at[s, pl.ds(stage * STAGE, STAGE)],
                                dst_v)

                for b in range(NBUF):
                    pltpu.async_copy(x_hbm.at[src_v.at[b]], rows[b],
                                     gsem.at[b])

                @pl.loop(0, STAGE, step=NBUF)
                def _(j):
                    for b in range(NBUF):
                        jj = j + b
                        pltpu.make_async_copy(x_hbm.at[src_v.at[jj]], rows[b],
                                              gsem.at[b]).wait()
                        pltpu.sync_copy(rows[b], acc_sh.at[dst_v.at[jj]],
                                        add=True)

                        @pl.when(jj + NBUF < STAGE)
                        def _():
                            pltpu.async_copy(x_hbm.at[src_v.at[jj + NBUF]],
                                             rows[b], gsem.at[b])

            plsc.subcore_barrier()
            pltpu.sync_copy(acc_sh.at[pl.ds(s * stripe, stripe)],
                            out_hbm.at[pl.ds(s * stripe, stripe)])

    return agg_kernel(x, src_r, dst_r, zeros_init)


def _tc_gin_mlp(x, agg, w1t, a1, c1, w2t, a2, c2):
    BLK = 2000

    def body(x_ref, g0_ref, w1_ref, a1_ref, c1_ref,
             w2_ref, a2_ref, c2_ref, o_ref):
        h = x_ref[...] + g0_ref[...]
        t = jnp.dot(h, w1_ref[...], preferred_element_type=jnp.float32,
                    precision=lax.Precision.HIGHEST)
        t = jnp.maximum(t * a1_ref[...] + c1_ref[...], 0.0)
        t = jnp.dot(t, w2_ref[...], preferred_element_type=jnp.float32,
                    precision=lax.Precision.HIGHEST)
        o_ref[...] = jnp.maximum(t * a2_ref[...] + c2_ref[...], 0.0)

    row_spec = pl.BlockSpec((BLK, D), lambda i: (i, 0))
    mat_spec = pl.BlockSpec((D, D), lambda i: (0, 0))
    vec_spec = pl.BlockSpec((1, D), lambda i: (0, 0))
    return pl.pallas_call(
        body,
        grid=(N // BLK,),
        in_specs=[row_spec, row_spec,
                  mat_spec, vec_spec, vec_spec,
                  mat_spec, vec_spec, vec_spec],
        out_specs=row_spec,
        out_shape=jax.ShapeDtypeStruct((N, D), jnp.float32),
    )(x, agg, w1t, a1, c1, w2t, a2, c2)


def _tc_set2set(x, batch2d, wih_t, whh_t, bias, lin_wt, lin_b):
    def body(x_ref, b_ref, wih0, wih1, wih2, wih3, whh0, whh1, whh2, whh3,
             b0, b1, b2, b3, lw_ref, lb_ref, o_ref):
        xv = x_ref[...]
        bcol = b_ref[...]
        gid = lax.broadcasted_iota(jnp.int32, (N, G), 1)
        msk = bcol == gid
        wihs = (wih0, wih1, wih2, wih3)
        whhs = (whh0, whh1, whh2, whh3)
        bs = (b0, b1, b2, b3)

        h = [jnp.zeros((G, D), jnp.float32) for _ in range(LSTM_LAYERS)]
        c = [jnp.zeros((G, D), jnp.float32) for _ in range(LSTM_LAYERS)]
        q_star = jnp.zeros((G, 2 * D), jnp.float32)

        for _ in range(STEPS):
            cur = q_star
            for l in range(LSTM_LAYERS):
                gates = (jnp.dot(cur, wihs[l][...],
                                 preferred_element_type=jnp.float32,
                                 precision=lax.Precision.HIGHEST)
                         + jnp.dot(h[l], whhs[l][...],
                                   preferred_element_type=jnp.float32,
                                   precision=lax.Precision.HIGHEST)
                         + bs[l][...])
                ig = jax.nn.sigmoid(gates[:, 0:D])
                fg = jax.nn.sigmoid(gates[:, D:2 * D])
                gg = jnp.tanh(gates[:, 2 * D:3 * D])
                og = jax.nn.sigmoid(gates[:, 3 * D:4 * D])
                c[l] = fg * c[l] + ig * gg
                h[l] = og * jnp.tanh(c[l])
                cur = h[l]
            q = cur
            e_mat = lax.dot_general(xv, q, (((1,), (1,)), ((), ())),
                                    preferred_element_type=jnp.float32,
                                    precision=lax.Precision.HIGHEST)
            e_masked = jnp.where(msk, e_mat, -1e30)
            e_max = jnp.max(e_masked, axis=0, keepdims=True)
            p = jnp.where(msk, jnp.exp(e_mat - e_max), 0.0)
            denom = jnp.sum(p, axis=0, keepdims=True)
            attn = p / (denom + 1e-16)
            r = lax.dot_general(attn, xv, (((0,), (0,)), ((), ())),
                                preferred_element_type=jnp.float32,
                                precision=lax.Precision.HIGHEST)
            q_star = jnp.concatenate([q, r], axis=1)

        o_ref[...] = (jnp.dot(q_star, lw_ref[...],
                              preferred_element_type=jnp.float32,
                              precision=lax.Precision.HIGHEST)
                      + lb_ref[...])

    full = lambda shape: pl.BlockSpec(shape, lambda: (0,) * len(shape))
    in_specs = ([full((N, D)), full((N, 1))]
                + [full(w.shape) for w in wih_t]
                + [full(w.shape) for w in whh_t]
                + [full(b.shape) for b in bias]
                + [full(lin_wt.shape), full(lin_b.shape)])
    return pl.pallas_call(
        body,
        in_specs=in_specs,
        out_specs=full((G, OUT)),
        out_shape=jax.ShapeDtypeStruct((G, OUT), jnp.float32),
    )(x, batch2d, *wih_t, *whh_t, *bias, lin_wt, lin_b)


def kernel(x, edge_index, batch, params):
    src = edge_index[0]
    dst = edge_index[1]
    pad = EPAD - E
    src_r = jnp.concatenate([src, jnp.zeros((pad,), jnp.int32)])
    dst_r = jnp.concatenate([dst, jnp.full((pad,), N, jnp.int32)])
    src_r = src_r.reshape(NS, EPAD // CHUNK // NS, CHUNK)
    dst_r = dst_r.reshape(NS, EPAD // CHUNK // NS, CHUNK)
    zeros_init = jnp.zeros((NPAD, D), jnp.float32)

    bn_k = 1.0 / jnp.sqrt(jnp.float32(1.0 + 1e-5))
    cur = x
    for p in params['gin']:
        agg = _sc_edge_agg(cur, src_r, dst_r, zeros_init)
        a1 = (p['g1'] * bn_k).reshape(1, D)
        c1 = (p['b1'] * p['g1'] * bn_k + p['be1']).reshape(1, D)
        a2 = (p['g2'] * bn_k).reshape(1, D)
        c2 = (p['b2'] * p['g2'] * bn_k + p['be2']).reshape(1, D)
        cur = _tc_gin_mlp(cur, agg[:N],
                          p['w1'].T, a1, c1, p['w2'].T, a2, c2)

    lstm = params['lstm']
    wih_t = [lp['wih'].T for lp in lstm]
    whh_t = [lp['whh'].T for lp in lstm]
    bias = [(lp['bih'] + lp['bhh']).reshape(1, 4 * D) for lp in lstm]
    batch2d = batch.reshape(N, 1)
    return _tc_set2set(cur, batch2d, wih_t, whh_t, bias,
                       params['lin_w'].T, params['lin_b'].reshape(1, OUT))


# speedup vs baseline: 2.2336x; 2.2336x over previous
import functools

import jax
import jax.numpy as jnp
from jax import lax
from jax.experimental import pallas as pl
from jax.experimental.pallas import tpu as pltpu
from jax.experimental.pallas import tpu_sc as plsc

N = 10000
E = 320000
D = 128
G = 64
OUT = 64
STEPS = 4
LSTM_LAYERS = 4

NC = 2
NS = 16
NW = NC * NS
CHUNK = 128
EPW_CHUNKS = 80
EPAD = NW * EPW_CHUNKS * CHUNK
NPAD = 10112


def _sc_edge_agg(x, src_r, dst_r, zeros_init):
    mesh = plsc.VectorSubcoreMesh(core_axis_name="c", subcore_axis_name="s")
    stripe = NPAD // NS
    NBUF = 2
    SUBC_CHUNKS = EPAD // CHUNK // NS
    STAGE = SUBC_CHUNKS // 4

    @functools.partial(
        pl.kernel,
        out_type=jax.ShapeDtypeStruct((NPAD, D), jnp.float32),
        mesh=mesh,
        scratch_types=[
            pltpu.VMEM((STAGE, CHUNK), jnp.int32),
            pltpu.VMEM((STAGE, CHUNK), jnp.int32),
            [pltpu.VMEM((CHUNK, D), jnp.float32) for _ in range(NBUF)],
            pltpu.VMEM_SHARED((NPAD, D), jnp.float32),
            pltpu.SemaphoreType.DMA((NBUF,)),
        ],
    )
    def agg_kernel(x_hbm, src_hbm, dst_hbm, z_hbm, out_hbm,
                   src_v, dst_v, rows, acc_sh, gsem):
        c = lax.axis_index("c")
        s = lax.axis_index("s")

        @pl.when(c == 0)
        def _():
            pltpu.sync_copy(z_hbm.at[pl.ds(s * stripe, stripe)],
                            acc_sh.at[pl.ds(s * stripe, stripe)])
            plsc.subcore_barrier()

            for stage in range(SUBC_CHUNKS // STAGE):
                pltpu.sync_copy(src_hbm.at[s, pl.ds(stage * STAGE, STAGE)],
                                src_v)
                pltpu.sync_copy(dst_hbm.at[s, pl.ds(stage * STAGE, STAGE)],
                                dst_v)

                for b in range(NBUF):
                    pltpu.async_copy(x_hbm.at[src_v.at[b]], rows[b],
                                     gsem.at[b])

                @pl.loop(0, STAGE, step=NBUF)
                def _(j):
                    for b in range(NBUF):
                        jj = j + b
                        pltpu.make_async_copy(x_hbm.at[src_v.at[jj]], rows[b],
                                              gsem.at[b]).wait()
                        pltpu.sync_copy(rows[b], acc_sh.at[dst_v.at[jj]],
                                        add=True)

                        @pl.when(jj + NBUF < STAGE)
                        def _():
                            pltpu.async_copy(x_hbm.at[src_v.at[jj + NBUF]],
                                             rows[b], gsem.at[b])

            plsc.subcore_barrier()
            pltpu.sync_copy(acc_sh.at[pl.ds(s * stripe, stripe)],
                            out_hbm.at[pl.ds(s * stripe, stripe)])

    return agg_kernel(x, src_r, dst_r, zeros_init)


def _tc_gin_mlp(x, agg, w1t, a1, c1, w2t, a2, c2):
    BLK = 2000

    def body(x_ref, g0_ref, w1_ref, a1_ref, c1_ref,
             w2_ref, a2_ref, c2_ref, o_ref):
        h = x_ref[...] + g0_ref[...]
        t = jnp.dot(h, w1_ref[...], preferred_element_type=jnp.float32,
                    precision=lax.Precision.HIGHEST)
        t = jnp.maximum(t * a1_ref[...] + c1_ref[...], 0.0)
        t = jnp.dot(t, w2_ref[...], preferred_element_type=jnp.float32,
                    precision=lax.Precision.HIGHEST)
        o_ref[...] = jnp.maximum(t * a2_ref[...] + c2_ref[...], 0.0)

    row_spec = pl.BlockSpec((BLK, D), lambda i: (i, 0))
    mat_spec = pl.BlockSpec((D, D), lambda i: (0, 0))
    vec_spec = pl.BlockSpec((1, D), lambda i: (0, 0))
    return pl.pallas_call(
        body,
        grid=(N // BLK,),
        in_specs=[row_spec, row_spec,
                  mat_spec, vec_spec, vec_spec,
                  mat_spec, vec_spec, vec_spec],
        out_specs=row_spec,
        out_shape=jax.ShapeDtypeStruct((N, D), jnp.float32),
    )(x, agg, w1t, a1, c1, w2t, a2, c2)


def _tc_set2set(x, batch2d, wih_t, whh_t, bias, lin_wt, lin_b):
    def body(x_ref, b_ref, wih0, wih1, wih2, wih3, whh0, whh1, whh2, whh3,
             b0, b1, b2, b3, lw_ref, lb_ref, o_ref):
        xv = x_ref[...]
        bcol = b_ref[...]
        gid = lax.broadcasted_iota(jnp.int32, (N, G), 1)
        msk = bcol == gid
        wihs = (wih0, wih1, wih2, wih3)
        whhs = (whh0, whh1, whh2, whh3)
        bs = (b0, b1, b2, b3)

        h = [jnp.zeros((G, D), jnp.float32) for _ in range(LSTM_LAYERS)]
        c = [jnp.zeros((G, D), jnp.float32) for _ in range(LSTM_LAYERS)]
        q_star = jnp.zeros((G, 2 * D), jnp.float32)

        for _ in range(STEPS):
            cur = q_star
            for l in range(LSTM_LAYERS):
                gates = (jnp.dot(cur, wihs[l][...],
                                 preferred_element_type=jnp.float32,
                                 precision=lax.Precision.HIGHEST)
                         + jnp.dot(h[l], whhs[l][...],
                                   preferred_element_type=jnp.float32,
                                   precision=lax.Precision.HIGHEST)
                         + bs[l][...])
                ig = jax.nn.sigmoid(gates[:, 0:D])
                fg = jax.nn.sigmoid(gates[:, D:2 * D])
                gg = jnp.tanh(gates[:, 2 * D:3 * D])
                og = jax.nn.sigmoid(gates[:, 3 * D:4 * D])
                c[l] = fg * c[l] + ig * gg
                h[l] = og * jnp.tanh(c[l])
                cur = h[l]
            q = cur
            e_mat = lax.dot_general(xv, q, (((1,), (1,)), ((), ())),
                                    preferred_element_type=jnp.float32,
                                    precision=lax.Precision.HIGHEST)
            e_masked = jnp.where(msk, e_mat, -1e30)
            e_max = jnp.max(e_masked, axis=0, keepdims=True)
            p = jnp.where(msk, jnp.exp(e_mat - e_max), 0.0)
            denom = jnp.sum(p, axis=0, keepdims=True)
            attn = p / (denom + 1e-16)
            r = lax.dot_general(attn, xv, (((0,), (0,)), ((), ())),
                                preferred_element_type=jnp.float32,
                                precision=lax.Precision.HIGHEST)
            q_star = jnp.concatenate([q, r], axis=1)

        o_ref[...] = (jnp.dot(q_star, lw_ref[...],
                              preferred_element_type=jnp.float32,
                              precision=lax.Precision.HIGHEST)
                      + lb_ref[...])

    full = lambda shape: pl.BlockSpec(shape, lambda: (0,) * len(shape))
    in_specs = ([full((N, D)), full((N, 1))]
                + [full(w.shape) for w in wih_t]
                + [full(w.shape) for w in whh_t]
                + [full(b.shape) for b in bias]
                + [full(lin_wt.shape), full(lin_b.shape)])
    return pl.pallas_call(
        body,
        in_specs=in_specs,
        out_specs=full((G, OUT)),
        out_shape=jax.ShapeDtypeStruct((G, OUT), jnp.float32),
    )(x, batch2d, *wih_t, *whh_t, *bias, lin_wt, lin_b)


def kernel(x, edge_index, batch, params):
    src = edge_index[0]
    dst = edge_index[1]
    pad = EPAD - E
    pad_dst = N + (jnp.arange(pad, dtype=jnp.int32) % (NPAD - N))
    pad_src = jnp.arange(pad, dtype=jnp.int32) % N
    src_r = jnp.concatenate([src, pad_src])
    dst_r = jnp.concatenate([dst, pad_dst])
    src_r = src_r.reshape(NS, EPAD // CHUNK // NS, CHUNK)
    dst_r = dst_r.reshape(NS, EPAD // CHUNK // NS, CHUNK)
    zeros_init = jnp.zeros((NPAD, D), jnp.float32)

    bn_k = 1.0 / jnp.sqrt(jnp.float32(1.0 + 1e-5))
    cur = x
    for p in params['gin']:
        agg = _sc_edge_agg(cur, src_r, dst_r, zeros_init)
        a1 = (p['g1'] * bn_k).reshape(1, D)
        c1 = (p['b1'] * p['g1'] * bn_k + p['be1']).reshape(1, D)
        a2 = (p['g2'] * bn_k).reshape(1, D)
        c2 = (p['b2'] * p['g2'] * bn_k + p['be2']).reshape(1, D)
        cur = _tc_gin_mlp(cur, agg[:N],
                          p['w1'].T, a1, c1, p['w2'].T, a2, c2)

    lstm = params['lstm']
    wih_t = [lp['wih'].T for lp in lstm]
    whh_t = [lp['whh'].T for lp in lstm]
    bias = [(lp['bih'] + lp['bhh']).reshape(1, 4 * D) for lp in lstm]
    batch2d = batch.reshape(N, 1)
    return _tc_set2set(cur, batch2d, wih_t, whh_t, bias,
                       params['lin_w'].T, params['lin_b'].reshape(1, OUT))


# speedup vs baseline: 3.2792x; 1.4681x over previous
import functools

import jax
import jax.numpy as jnp
from jax import lax
from jax.experimental import pallas as pl
from jax.experimental.pallas import tpu as pltpu
from jax.experimental.pallas import tpu_sc as plsc

N = 10000
E = 320000
D = 128
G = 64
OUT = 64
STEPS = 4
LSTM_LAYERS = 4

NC = 2
NS = 16
NW = NC * NS
CHUNK = 128
EPW_CHUNKS = 80
EPAD = NW * EPW_CHUNKS * CHUNK
NPAD = 10112


def _sc_edge_agg(x, src_r, dst_r, zeros_init):
    mesh = plsc.VectorSubcoreMesh(core_axis_name="c", subcore_axis_name="s")
    stripe = NPAD // NS
    NBUF = 2
    HALF = EPW_CHUNKS // 2

    @functools.partial(
        pl.kernel,
        out_type=jax.ShapeDtypeStruct((NC, NPAD, D), jnp.float32),
        mesh=mesh,
        scratch_types=[
            pltpu.VMEM((HALF, CHUNK), jnp.int32),
            pltpu.VMEM((HALF, CHUNK), jnp.int32),
            [pltpu.VMEM((CHUNK, D), jnp.float32) for _ in range(NBUF)],
            pltpu.VMEM_SHARED((NPAD, D), jnp.float32),
            pltpu.SemaphoreType.DMA((NBUF,)),
        ],
    )
    def agg_kernel(x_hbm, src_hbm, dst_hbm, z_hbm, out_hbm,
                   src_v, dst_v, rows, acc_sh, gsem):
        c = lax.axis_index("c")
        s = lax.axis_index("s")
        wid = c * NS + s
        pltpu.sync_copy(z_hbm.at[pl.ds(s * stripe, stripe)],
                        acc_sh.at[pl.ds(s * stripe, stripe)])
        plsc.subcore_barrier()

        for half in range(2):
            pltpu.sync_copy(src_hbm.at[wid, pl.ds(half * HALF, HALF)], src_v)
            pltpu.sync_copy(dst_hbm.at[wid, pl.ds(half * HALF, HALF)], dst_v)

            for b in range(NBUF):
                pltpu.async_copy(x_hbm.at[src_v.at[b]], rows[b], gsem.at[b])

            @pl.loop(0, HALF, step=NBUF)
            def _(j):
                for b in range(NBUF):
                    jj = j + b
                    pltpu.make_async_copy(x_hbm.at[src_v.at[jj]], rows[b],
                                          gsem.at[b]).wait()
                    pltpu.sync_copy(rows[b], acc_sh.at[dst_v.at[jj]],
                                    add=True)

                    @pl.when(jj + NBUF < HALF)
                    def _():
                        pltpu.async_copy(x_hbm.at[src_v.at[jj + NBUF]],
                                         rows[b], gsem.at[b])

        plsc.subcore_barrier()
        pltpu.sync_copy(acc_sh.at[pl.ds(s * stripe, stripe)],
                        out_hbm.at[c, pl.ds(s * stripe, stripe)])

    return agg_kernel(x, src_r, dst_r, zeros_init)


def _tc_gin_mlp(x, agg0, agg1, w1t, a1, c1, w2t, a2, c2):
    BLK = 2000

    def body(x_ref, g0_ref, g1_ref, w1_ref, a1_ref, c1_ref,
             w2_ref, a2_ref, c2_ref, o_ref):
        h = x_ref[...] + g0_ref[...] + g1_ref[...]
        t = jnp.dot(h, w1_ref[...], preferred_element_type=jnp.float32,
                    precision=lax.Precision.HIGHEST)
        t = jnp.maximum(t * a1_ref[...] + c1_ref[...], 0.0)
        t = jnp.dot(t, w2_ref[...], preferred_element_type=jnp.float32,
                    precision=lax.Precision.HIGHEST)
        o_ref[...] = jnp.maximum(t * a2_ref[...] + c2_ref[...], 0.0)

    row_spec = pl.BlockSpec((BLK, D), lambda i: (i, 0))
    mat_spec = pl.BlockSpec((D, D), lambda i: (0, 0))
    vec_spec = pl.BlockSpec((1, D), lambda i: (0, 0))
    return pl.pallas_call(
        body,
        grid=(N // BLK,),
        in_specs=[row_spec, row_spec, row_spec,
                  mat_spec, vec_spec, vec_spec,
                  mat_spec, vec_spec, vec_spec],
        out_specs=row_spec,
        out_shape=jax.ShapeDtypeStruct((N, D), jnp.float32),
    )(x, agg0, agg1, w1t, a1, c1, w2t, a2, c2)


def _tc_set2set(x, batch2d, wih_t, whh_t, bias, lin_wt, lin_b):
    def body(x_ref, b_ref, wih0, wih1, wih2, wih3, whh0, whh1, whh2, whh3,
             b0, b1, b2, b3, lw_ref, lb_ref, o_ref):
        xv = x_ref[...]
        bcol = b_ref[...]
        gid = lax.broadcasted_iota(jnp.int32, (N, G), 1)
        msk = bcol == gid
        wihs = (wih0, wih1, wih2, wih3)
        whhs = (whh0, whh1, whh2, whh3)
        bs = (b0, b1, b2, b3)

        h = [jnp.zeros((G, D), jnp.float32) for _ in range(LSTM_LAYERS)]
        c = [jnp.zeros((G, D), jnp.float32) for _ in range(LSTM_LAYERS)]
        q_star = jnp.zeros((G, 2 * D), jnp.float32)

        for _ in range(STEPS):
            cur = q_star
            for l in range(LSTM_LAYERS):
                gates = (jnp.dot(cur, wihs[l][...],
                                 preferred_element_type=jnp.float32,
                                 precision=lax.Precision.HIGHEST)
                         + jnp.dot(h[l], whhs[l][...],
                                   preferred_element_type=jnp.float32,
                                   precision=lax.Precision.HIGHEST)
                         + bs[l][...])
                ig = jax.nn.sigmoid(gates[:, 0:D])
                fg = jax.nn.sigmoid(gates[:, D:2 * D])
                gg = jnp.tanh(gates[:, 2 * D:3 * D])
                og = jax.nn.sigmoid(gates[:, 3 * D:4 * D])
                c[l] = fg * c[l] + ig * gg
                h[l] = og * jnp.tanh(c[l])
                cur = h[l]
            q = cur
            e_mat = lax.dot_general(xv, q, (((1,), (1,)), ((), ())),
                                    preferred_element_type=jnp.float32,
                                    precision=lax.Precision.HIGHEST)
            e_masked = jnp.where(msk, e_mat, -1e30)
            e_max = jnp.max(e_masked, axis=0, keepdims=True)
            p = jnp.where(msk, jnp.exp(e_mat - e_max), 0.0)
            denom = jnp.sum(p, axis=0, keepdims=True)
            attn = p / (denom + 1e-16)
            r = lax.dot_general(attn, xv, (((0,), (0,)), ((), ())),
                                preferred_element_type=jnp.float32,
                                precision=lax.Precision.HIGHEST)
            q_star = jnp.concatenate([q, r], axis=1)

        o_ref[...] = (jnp.dot(q_star, lw_ref[...],
                              preferred_element_type=jnp.float32,
                              precision=lax.Precision.HIGHEST)
                      + lb_ref[...])

    full = lambda shape: pl.BlockSpec(shape, lambda: (0,) * len(shape))
    in_specs = ([full((N, D)), full((N, 1))]
                + [full(w.shape) for w in wih_t]
                + [full(w.shape) for w in whh_t]
                + [full(b.shape) for b in bias]
                + [full(lin_wt.shape), full(lin_b.shape)])
    return pl.pallas_call(
        body,
        in_specs=in_specs,
        out_specs=full((G, OUT)),
        out_shape=jax.ShapeDtypeStruct((G, OUT), jnp.float32),
    )(x, batch2d, *wih_t, *whh_t, *bias, lin_wt, lin_b)


def kernel(x, edge_index, batch, params):
    src = edge_index[0]
    dst = edge_index[1]
    pad = EPAD - E
    pad_dst = N + (jnp.arange(pad, dtype=jnp.int32) % (NPAD - N))
    pad_src = jnp.arange(pad, dtype=jnp.int32) % N
    src_r = jnp.concatenate([src, pad_src])
    dst_r = jnp.concatenate([dst, pad_dst])
    src_r = src_r.reshape(NW, EPW_CHUNKS, CHUNK)
    dst_r = dst_r.reshape(NW, EPW_CHUNKS, CHUNK)
    zeros_init = jnp.zeros((NPAD, D), jnp.float32)

    bn_k = 1.0 / jnp.sqrt(jnp.float32(1.0 + 1e-5))
    cur = x
    for p in params['gin']:
        agg = _sc_edge_agg(cur, src_r, dst_r, zeros_init)
        a1 = (p['g1'] * bn_k).reshape(1, D)
        c1 = (p['b1'] * p['g1'] * bn_k + p['be1']).reshape(1, D)
        a2 = (p['g2'] * bn_k).reshape(1, D)
        c2 = (p['b2'] * p['g2'] * bn_k + p['be2']).reshape(1, D)
        cur = _tc_gin_mlp(cur, agg[0, :N], agg[1, :N],
                          p['w1'].T, a1, c1, p['w2'].T, a2, c2)

    lstm = params['lstm']
    wih_t = [lp['wih'].T for lp in lstm]
    whh_t = [lp['whh'].T for lp in lstm]
    bias = [(lp['bih'] + lp['bhh']).reshape(1, 4 * D) for lp in lstm]
    batch2d = batch.reshape(N, 1)
    return _tc_set2set(cur, batch2d, wih_t, whh_t, bias,
                       params['lin_w'].T, params['lin_b'].reshape(1, OUT))
